# baseline (speedup 1.0000x reference)
import functools

import jax
import jax.numpy as jnp
import numpy as np
from jax import lax
from jax.experimental import pallas as pl
from jax.experimental.pallas import tpu as pltpu
from jax.experimental.pallas import tpu_sc as plsc

_VOCAB_SIZE = 128
_PATCH_SIZE = 16
_LANES = 16

_ROT_A = (13, 15, 26, 6)
_ROT_B = (17, 29, 16, 24)
_PARITY = np.uint32(0x1BD11BDA)


def _np_rotl(x, r):
    return ((x << np.uint32(r)) | (x >> np.uint32(32 - r))).astype(np.uint32)


def _np_threefry2x32(k0, k1, x0, x1):
    ks = [np.uint32(k0), np.uint32(k1),
          np.uint32(np.uint32(k0) ^ np.uint32(k1) ^ _PARITY)]
    x = [(x0 + ks[0]).astype(np.uint32), (x1 + ks[1]).astype(np.uint32)]
    for i in range(5):
        for r in (_ROT_A, _ROT_B)[i % 2]:
            x[0] = (x[0] + x[1]).astype(np.uint32)
            x[1] = _np_rotl(x[1], r)
            x[1] = x[1] ^ x[0]
        x[0] = (x[0] + ks[(i + 1) % 3]).astype(np.uint32)
        x[1] = (x[1] + ks[(i + 2) % 3] + np.uint32(i + 1)).astype(np.uint32)
    return x


@functools.lru_cache(maxsize=None)
def _split_keys():
    b1, b2 = _np_threefry2x32(np.uint32(0), np.uint32(42),
                              np.zeros(2, np.uint32),
                              np.arange(2, dtype=np.uint32))
    return (int(b1[0]), int(b2[0])), (int(b1[1]), int(b2[1]))


def _sel3(m, a, b, c):
    return jnp.where(m == 0, a, jnp.where(m == 1, b, c))


def _sc_threefry_bits2(k0, k1, cnt_a, cnt_b):
    k2 = k0 ^ k1 ^ jnp.uint32(_PARITY)
    a0 = jnp.zeros((_LANES,), jnp.uint32) + k0
    b0 = a0
    a1 = cnt_a + k1
    b1 = cnt_b + k1

    def group(i, carry):
        a0, a1, b0, b1 = carry
        odd = (i % 2).astype(jnp.uint32)
        for ra, rb in zip(_ROT_A, _ROT_B):
            d = jnp.where(odd == 0, jnp.uint32(ra), jnp.uint32(rb))
            dn = jnp.uint32(32) - d
            a0 = a0 + a1
            b0 = b0 + b1
            a1 = (a1 << d) | (a1 >> dn)
            b1 = (b1 << d) | (b1 >> dn)
            a1 = a1 ^ a0
            b1 = b1 ^ b0
        m1 = ((i + 1) % 3).astype(jnp.uint32)
        m2 = ((i + 2) % 3).astype(jnp.uint32)
        j1 = _sel3(m1, k0, k1, k2)
        j2 = _sel3(m2, k0, k1, k2) + (i + 1).astype(jnp.uint32)
        return (a0 + j1, a1 + j2, b0 + j1, b1 + j2)

    a0, a1, b0, b1 = lax.fori_loop(0, 5, group, (a0, a1, b0, b1))
    return a0 ^ a1, b0 ^ b1


@functools.lru_cache(maxsize=None)
def _make_sc_kernel(batch_size: int, n_rows: int, n_cols: int):
    info = plsc.get_sparse_core_info()
    ns = info.num_subcores
    nc = 1
    nw = nc * ns
    assert batch_size % nw == 0
    n_batch_per_w = batch_size // nw
    assert n_batch_per_w <= 2
    assert _VOCAB_SIZE % n_rows == 0 and _VOCAB_SIZE % n_cols == 0
    w_row, w_col = _VOCAB_SIZE // n_rows, _VOCAB_SIZE // n_cols
    assert w_row & (w_row - 1) == 0 and w_col & (w_col - 1) == 0
    lg_row, lg_col = w_row.bit_length() - 1, w_col.bit_length() - 1
    assert n_rows % _LANES == 0 and n_cols % _LANES == 0
    (kr0, kr1), (kc0, kc1) = _split_keys()
    mesh = plsc.VectorSubcoreMesh(core_axis_name="c", subcore_axis_name="s",
                                  num_cores=nc)
    n_row_chunks = n_rows // _LANES
    n_col_chunks = n_cols // _LANES
    assert n_row_chunks == 2 and n_col_chunks == 2
    chunks_per_batch = n_row_chunks + n_col_chunks
    n_chunks = n_batch_per_w * chunks_per_batch

    @functools.partial(
        pl.kernel,
        mesh=mesh,
        compiler_params=pltpu.CompilerParams(needs_layout_passes=False),
        out_type=jax.ShapeDtypeStruct((batch_size, n_rows, n_cols), jnp.float32),
        scratch_types=[
            pltpu.VMEM((_VOCAB_SIZE,), jnp.float32),
            pltpu.VMEM((_VOCAB_SIZE,), jnp.float32),
            pltpu.VMEM((n_chunks * _LANES,), jnp.uint32),
            pltpu.VMEM((n_rows + _LANES,), jnp.float32),
            pltpu.VMEM((2, n_rows, n_cols), jnp.float32),
            pltpu.SemaphoreType.DMA,
            pltpu.SemaphoreType.DMA,
        ],
    )
    def sc_kernel(rt_hbm, ct_hbm, out_hbm, rt_v, ct_v, bits_v, rvals_v,
                  out_v, sem, out_sem):
        wid = lax.axis_index("s") * nc + lax.axis_index("c")
        lane = lax.iota(jnp.int32, _LANES)
        c1 = pltpu.async_copy(rt_hbm, rt_v, sem)
        c2 = pltpu.async_copy(ct_hbm, ct_v, sem)

        def hash_pair(p, _):
            t = p // 2
            is_col = (p % 2) == 1
            b = wid + nw * t
            n_pos = jnp.where(is_col, n_cols, n_rows)
            base = (b * n_pos + lane).astype(jnp.uint32)
            k0 = jnp.where(is_col, jnp.uint32(kc0), jnp.uint32(kr0))
            k1 = jnp.where(is_col, jnp.uint32(kc1), jnp.uint32(kr1))
            bits_a, bits_b = _sc_threefry_bits2(
                k0, k1, base, base + jnp.uint32(_LANES)
            )
            off = p * 2 * _LANES
            bits_v[pl.ds(off, _LANES)] = bits_a
            bits_v[pl.ds(off + _LANES, _LANES)] = bits_b
            return 0

        lax.fori_loop(0, n_chunks // 2, hash_pair, 0)
        c1.wait()
        c2.wait()

        out_copies = []
        for t in range(n_batch_per_w):
            base = t * chunks_per_batch * _LANES
            cvals = []
            for j in range(0, n_cols, _LANES):
                bits = bits_v[pl.ds(base + n_rows + j, _LANES)]
                frac = ((bits >> jnp.uint32(32 - lg_col)).astype(jnp.int32)
                        if lg_col else 0)
                cidx = w_col * (j + lane) + frac
                cvals.append(plsc.load_gather(ct_v, [cidx]))
            for j in range(0, n_rows, _LANES):
                bits = bits_v[pl.ds(base + j, _LANES)]
                frac = ((bits >> jnp.uint32(32 - lg_row)).astype(jnp.int32)
                        if lg_row else 0)
                ridx = w_row * (j + lane) + frac
                rvals_v[pl.ds(_LANES + j, _LANES)] = plsc.load_gather(
                    rt_v, [ridx]
                )

            half = n_rows // 2

            def out_rows(r, _):
                zero16 = jnp.zeros((_LANES,), jnp.int32)
                rv0 = plsc.load_gather(rvals_v, [zero16 + (_LANES + r)])
                rv1 = plsc.load_gather(rvals_v, [zero16 + (_LANES + half + r)])
                for j in range(0, n_cols, _LANES):
                    cv = cvals[j // _LANES]
                    out_v[t % 2, r, pl.ds(j, _LANES)] = rv0 + cv
                    out_v[t % 2, half + r, pl.ds(j, _LANES)] = rv1 + cv
                return 0

            lax.fori_loop(0, half, out_rows, 0)
            out_copies.append(
                pltpu.async_copy(out_v.at[t % 2], out_hbm.at[wid + nw * t],
                                 out_sem)
            )
            if t >= 1:
                out_copies[t - 1].wait()
        out_copies[-1].wait()

    return sc_kernel


def kernel(images, row_table, col_table):
    batch_size, _, height, width = images.shape
    n_rows = height // _PATCH_SIZE
    n_cols = width // _PATCH_SIZE
    sc = _make_sc_kernel(batch_size, n_rows, n_cols)
    out = sc(row_table.reshape(_VOCAB_SIZE), col_table.reshape(_VOCAB_SIZE))
    return out.reshape(batch_size, 1, n_rows, n_cols)


# speedup vs baseline: 1.0103x; 1.0103x over previous
import functools

import jax
import jax.numpy as jnp
import numpy as np
from jax import lax
from jax.experimental import pallas as pl
from jax.experimental.pallas import tpu as pltpu
from jax.experimental.pallas import tpu_sc as plsc

_VOCAB_SIZE = 128
_PATCH_SIZE = 16
_LANES = 16

_ROT_A = (13, 15, 26, 6)
_ROT_B = (17, 29, 16, 24)
_PARITY = np.uint32(0x1BD11BDA)


def _np_rotl(x, r):
    return ((x << np.uint32(r)) | (x >> np.uint32(32 - r))).astype(np.uint32)


def _np_threefry2x32(k0, k1, x0, x1):
    ks = [np.uint32(k0), np.uint32(k1),
          np.uint32(np.uint32(k0) ^ np.uint32(k1) ^ _PARITY)]
    x = [(x0 + ks[0]).astype(np.uint32), (x1 + ks[1]).astype(np.uint32)]
    for i in range(5):
        for r in (_ROT_A, _ROT_B)[i % 2]:
            x[0] = (x[0] + x[1]).astype(np.uint32)
            x[1] = _np_rotl(x[1], r)
            x[1] = x[1] ^ x[0]
        x[0] = (x[0] + ks[(i + 1) % 3]).astype(np.uint32)
        x[1] = (x[1] + ks[(i + 2) % 3] + np.uint32(i + 1)).astype(np.uint32)
    return x


@functools.lru_cache(maxsize=None)
def _split_keys():
    b1, b2 = _np_threefry2x32(np.uint32(0), np.uint32(42),
                              np.zeros(2, np.uint32),
                              np.arange(2, dtype=np.uint32))
    return (int(b1[0]), int(b2[0])), (int(b1[1]), int(b2[1]))


def _sel3(m, a, b, c):
    return jnp.where(m == 0, a, jnp.where(m == 1, b, c))


def _sc_threefry_bits2(k0, k1, cnt_a, cnt_b):
    k2 = k0 ^ k1 ^ jnp.uint32(_PARITY)
    a0 = jnp.zeros((_LANES,), jnp.uint32) + k0
    b0 = a0
    a1 = cnt_a + k1
    b1 = cnt_b + k1

    def group(i, carry):
        a0, a1, b0, b1 = carry
        odd = (i % 2).astype(jnp.uint32)
        for ra, rb in zip(_ROT_A, _ROT_B):
            d = jnp.where(odd == 0, jnp.uint32(ra), jnp.uint32(rb))
            dn = jnp.uint32(32) - d
            a0 = a0 + a1
            b0 = b0 + b1
            a1 = (a1 << d) | (a1 >> dn)
            b1 = (b1 << d) | (b1 >> dn)
            a1 = a1 ^ a0
            b1 = b1 ^ b0
        m1 = ((i + 1) % 3).astype(jnp.uint32)
        m2 = ((i + 2) % 3).astype(jnp.uint32)
        j1 = _sel3(m1, k0, k1, k2)
        j2 = _sel3(m2, k0, k1, k2) + (i + 1).astype(jnp.uint32)
        return (a0 + j1, a1 + j2, b0 + j1, b1 + j2)

    a0, a1, b0, b1 = lax.fori_loop(0, 5, group, (a0, a1, b0, b1))
    return a0 ^ a1, b0 ^ b1


def _sc_threefry_bits4(kr, kc, cra, crb, cca, ccb):
    kr0, kr1 = kr
    kc0, kc1 = kc
    kr2 = kr0 ^ kr1 ^ jnp.uint32(_PARITY)
    kc2 = kc0 ^ kc1 ^ jnp.uint32(_PARITY)
    zeros = jnp.zeros((_LANES,), jnp.uint32)
    ra0, rb0 = zeros + kr0, zeros + kr0
    ca0, cb0 = zeros + kc0, zeros + kc0
    ra1, rb1 = cra + kr1, crb + kr1
    ca1, cb1 = cca + kc1, ccb + kc1

    def group(i, carry):
        ra0, ra1, rb0, rb1, ca0, ca1, cb0, cb1 = carry
        odd = (i % 2).astype(jnp.uint32)
        for rot_a, rot_b in zip(_ROT_A, _ROT_B):
            d = jnp.where(odd == 0, jnp.uint32(rot_a), jnp.uint32(rot_b))
            dn = jnp.uint32(32) - d
            ra0 = ra0 + ra1
            rb0 = rb0 + rb1
            ca0 = ca0 + ca1
            cb0 = cb0 + cb1
            ra1 = (ra1 << d) | (ra1 >> dn)
            rb1 = (rb1 << d) | (rb1 >> dn)
            ca1 = (ca1 << d) | (ca1 >> dn)
            cb1 = (cb1 << d) | (cb1 >> dn)
            ra1 = ra1 ^ ra0
            rb1 = rb1 ^ rb0
            ca1 = ca1 ^ ca0
            cb1 = cb1 ^ cb0
        m1 = ((i + 1) % 3).astype(jnp.uint32)
        m2 = ((i + 2) % 3).astype(jnp.uint32)
        inc = (i + 1).astype(jnp.uint32)
        jr1 = _sel3(m1, kr0, kr1, kr2)
        jr2 = _sel3(m2, kr0, kr1, kr2) + inc
        jc1 = _sel3(m1, kc0, kc1, kc2)
        jc2 = _sel3(m2, kc0, kc1, kc2) + inc
        return (ra0 + jr1, ra1 + jr2, rb0 + jr1, rb1 + jr2,
                ca0 + jc1, ca1 + jc2, cb0 + jc1, cb1 + jc2)

    ra0, ra1, rb0, rb1, ca0, ca1, cb0, cb1 = lax.fori_loop(
        0, 5, group, (ra0, ra1, rb0, rb1, ca0, ca1, cb0, cb1)
    )
    return ra0 ^ ra1, rb0 ^ rb1, ca0 ^ ca1, cb0 ^ cb1


@functools.lru_cache(maxsize=None)
def _make_sc_kernel(batch_size: int, n_rows: int, n_cols: int):
    info = plsc.get_sparse_core_info()
    ns = info.num_subcores
    nc = 1
    nw = nc * ns
    assert batch_size % nw == 0
    n_batch_per_w = batch_size // nw
    assert n_batch_per_w <= 2
    assert _VOCAB_SIZE % n_rows == 0 and _VOCAB_SIZE % n_cols == 0
    w_row, w_col = _VOCAB_SIZE // n_rows, _VOCAB_SIZE // n_cols
    assert w_row & (w_row - 1) == 0 and w_col & (w_col - 1) == 0
    lg_row, lg_col = w_row.bit_length() - 1, w_col.bit_length() - 1
    assert n_rows % _LANES == 0 and n_cols % _LANES == 0
    (kr0, kr1), (kc0, kc1) = _split_keys()
    mesh = plsc.VectorSubcoreMesh(core_axis_name="c", subcore_axis_name="s",
                                  num_cores=nc)
    n_row_chunks = n_rows // _LANES
    n_col_chunks = n_cols // _LANES
    assert n_row_chunks == 2 and n_col_chunks == 2
    chunks_per_batch = n_row_chunks + n_col_chunks
    n_chunks = n_batch_per_w * chunks_per_batch

    @functools.partial(
        pl.kernel,
        mesh=mesh,
        compiler_params=pltpu.CompilerParams(needs_layout_passes=False),
        out_type=jax.ShapeDtypeStruct((batch_size, n_rows, n_cols), jnp.float32),
        scratch_types=[
            pltpu.VMEM((_VOCAB_SIZE,), jnp.float32),
            pltpu.VMEM((_VOCAB_SIZE,), jnp.float32),
            pltpu.VMEM((n_chunks * _LANES,), jnp.uint32),
            pltpu.VMEM((2, n_rows, n_cols), jnp.float32),
            pltpu.SemaphoreType.DMA,
            pltpu.SemaphoreType.DMA,
        ],
    )
    def sc_kernel(rt_hbm, ct_hbm, out_hbm, rt_v, ct_v, bits_v,
                  out_v, sem, out_sem):
        wid = lax.axis_index("s") * nc + lax.axis_index("c")
        lane = lax.iota(jnp.int32, _LANES)
        c1 = pltpu.async_copy(rt_hbm, rt_v, sem)
        c2 = pltpu.async_copy(ct_hbm, ct_v, sem)

        def hash_batch(t, _):
            b = wid + nw * t
            rbase = (b * n_rows + lane).astype(jnp.uint32)
            cbase = (b * n_cols + lane).astype(jnp.uint32)
            bra, brb, bca, bcb = _sc_threefry_bits4(
                (jnp.uint32(kr0), jnp.uint32(kr1)),
                (jnp.uint32(kc0), jnp.uint32(kc1)),
                rbase, rbase + jnp.uint32(_LANES),
                cbase, cbase + jnp.uint32(_LANES),
            )
            off = t * chunks_per_batch * _LANES
            bits_v[pl.ds(off, _LANES)] = bra
            bits_v[pl.ds(off + _LANES, _LANES)] = brb
            bits_v[pl.ds(off + 2 * _LANES, _LANES)] = bca
            bits_v[pl.ds(off + 3 * _LANES, _LANES)] = bcb
            return 0

        lax.fori_loop(0, n_batch_per_w, hash_batch, 0)
        c1.wait()
        c2.wait()

        out_copies = []
        for t in range(n_batch_per_w):
            base = t * chunks_per_batch * _LANES
            cvals, rvals = [], []
            for j in range(0, n_cols, _LANES):
                bits = bits_v[pl.ds(base + n_rows + j, _LANES)]
                frac = ((bits >> jnp.uint32(32 - lg_col)).astype(jnp.int32)
                        if lg_col else 0)
                cidx = w_col * (j + lane) + frac
                cvals.append(plsc.load_gather(ct_v, [cidx]))
            for j in range(0, n_rows, _LANES):
                bits = bits_v[pl.ds(base + j, _LANES)]
                frac = ((bits >> jnp.uint32(32 - lg_row)).astype(jnp.int32)
                        if lg_row else 0)
                ridx = w_row * (j + lane) + frac
                rvals.append(plsc.load_gather(rt_v, [ridx]))

            def out_rows(r, _):
                splat = jnp.zeros((_LANES,), jnp.int32) + r
                for k, rvec in enumerate(rvals):
                    rv = rvec.at[splat].get(mode="promise_in_bounds")
                    for j in range(0, n_cols, _LANES):
                        out_v[t % 2, k * _LANES + r, pl.ds(j, _LANES)] = (
                            rv + cvals[j // _LANES]
                        )
                return 0

            lax.fori_loop(0, _LANES, out_rows, 0)
            out_copies.append(
                pltpu.async_copy(out_v.at[t % 2], out_hbm.at[wid + nw * t],
                                 out_sem)
            )
            if t >= 1:
                out_copies[t - 1].wait()
        out_copies[-1].wait()

    return sc_kernel


def kernel(images, row_table, col_table):
    batch_size, _, height, width = images.shape
    n_rows = height // _PATCH_SIZE
    n_cols = width // _PATCH_SIZE
    sc = _make_sc_kernel(batch_size, n_rows, n_cols)
    out = sc(row_table.reshape(_VOCAB_SIZE), col_table.reshape(_VOCAB_SIZE))
    return out.reshape(batch_size, 1, n_rows, n_cols)
